# baseline (device time: 506431 ns/iter reference)
import jax
import jax.numpy as jnp
from jax import lax
from jax.experimental import pallas as pl
from jax.experimental.pallas import tpu as pltpu

N_DEV = 4
S = 4096
D = 1024
H = 8
DH = 128
BLK = 512
EPS = 1e-5
SCALE = 0.08838834764831843


def _ln_mod_matmul3(x2, scale_v, shift_v, Wa, Wb, Wc):

    def body(x_ref, sc_ref, sh_ref, wa_ref, wb_ref, wc_ref, a_ref, b_ref, c_ref):
        xb = x_ref[...]
        m = jnp.mean(xb, axis=1, keepdims=True)
        xc = xb - m
        var = jnp.mean(xc * xc, axis=1, keepdims=True)
        xn = xc * lax.rsqrt(var + EPS)
        xm = (xn * (1.0 + sc_ref[...]) + sh_ref[...]).astype(jnp.bfloat16)
        a_ref[...] = (
            jnp.dot(xm, wa_ref[...], preferred_element_type=jnp.float32) * SCALE
        ).astype(jnp.bfloat16)
        b_ref[...] = jnp.dot(
            xm, wb_ref[...], preferred_element_type=jnp.float32
        ).astype(jnp.bfloat16)
        c_ref[...] = jnp.dot(
            xm, wc_ref[...], preferred_element_type=jnp.float32
        ).astype(jnp.bfloat16)

    vec_spec = pl.BlockSpec((1, D), lambda i: (0, 0))
    w_spec = pl.BlockSpec((D, D), lambda i: (0, 0))
    seq_spec = pl.BlockSpec((BLK, D), lambda i: (i, 0))
    out = jax.ShapeDtypeStruct((S, D), jnp.bfloat16)
    return pl.pallas_call(
        body,
        grid=(S // BLK,),
        in_specs=[seq_spec, vec_spec, vec_spec, w_spec, w_spec, w_spec],
        out_specs=(seq_spec, seq_spec, seq_spec),
        out_shape=(out, out, out),
    )(x2, scale_v, shift_v, Wa, Wb, Wc)


def _attention(Q, K, V):

    KB = 1024

    def body(q_ref, k_ref, v_ref, o_ref):
        q = q_ref[...]
        acc = jnp.zeros((BLK, DH), jnp.float32)
        l = jnp.zeros((BLK, 1), jnp.float32)
        for j in range(S // KB):
            k = k_ref[j * KB:(j + 1) * KB, :]
            s = lax.dot_general(
                q, k, (((1,), (1,)), ((), ())),
                preferred_element_type=jnp.float32,
            )
            p = jnp.exp(s)
            l = l + jnp.sum(p, axis=1, keepdims=True)
            acc = acc + jnp.dot(
                p.astype(jnp.bfloat16),
                v_ref[j * KB:(j + 1) * KB, :],
                preferred_element_type=jnp.float32,
            )
        o_ref[...] = (acc / l).astype(jnp.bfloat16)

    q_spec = pl.BlockSpec((BLK, DH), lambda h, qb: (qb, h))
    kv_spec = pl.BlockSpec((S, DH), lambda h, qb: (0, h))
    return pl.pallas_call(
        body,
        grid=(H, S // BLK),
        in_specs=[q_spec, kv_spec, kv_spec],
        out_specs=q_spec,
        out_shape=jax.ShapeDtypeStruct((S, H * DH), jnp.bfloat16),
    )(Q, K, V)


def _matmul(A, B):

    def body(a_ref, b_ref, o_ref):
        o_ref[...] = jnp.dot(
            a_ref[...], b_ref[...], preferred_element_type=jnp.float32
        ).astype(jnp.bfloat16)

    return pl.pallas_call(
        body,
        grid=(S // BLK,),
        in_specs=[
            pl.BlockSpec((BLK, D), lambda i: (i, 0)),
            pl.BlockSpec((D, D), lambda i: (0, 0)),
        ],
        out_specs=pl.BlockSpec((BLK, D), lambda i: (i, 0)),
        out_shape=jax.ShapeDtypeStruct((S, D), jnp.bfloat16),
    )(A, B)


def _ln_mod_ffn_partial(x2, scale_v, shift_v, W1, W2):

    def body(x_ref, sc_ref, sh_ref, w1_ref, w2_ref, o_ref):
        xb = x_ref[...]
        m = jnp.mean(xb, axis=1, keepdims=True)
        xc = xb - m
        var = jnp.mean(xc * xc, axis=1, keepdims=True)
        xn = xc * lax.rsqrt(var + EPS)
        xm = (xn * (1.0 + sc_ref[...]) + sh_ref[...]).astype(jnp.bfloat16)
        h = jnp.dot(xm, w1_ref[...], preferred_element_type=jnp.float32)
        h = (h * jax.nn.sigmoid(h)).astype(jnp.bfloat16)
        o_ref[...] = jnp.dot(
            h, w2_ref[...], preferred_element_type=jnp.float32
        ).astype(jnp.bfloat16)

    vec_spec = pl.BlockSpec((1, D), lambda i: (0, 0))
    w_spec = pl.BlockSpec((D, D), lambda i: (0, 0))
    seq_spec = pl.BlockSpec((BLK, D), lambda i: (i, 0))
    return pl.pallas_call(
        body,
        grid=(S // BLK,),
        in_specs=[seq_spec, vec_spec, vec_spec, w_spec, w_spec],
        out_specs=seq_spec,
        out_shape=jax.ShapeDtypeStruct((S, D), jnp.bfloat16),
    )(x2, scale_v, shift_v, W1, W2)


def _allreduce_residual(partial, resid, gate, collective_id):

    C = S // N_DEV
    HD = D // 2

    def body(p_ref, r_ref, g_ref, o_ref, rbR, rbL, agR, agL, ssR, rsR, ssL, rsL):
        my = lax.axis_index("i")
        left = lax.rem(my + N_DEV - 1, N_DEV)
        right = lax.rem(my + 1, N_DEV)

        barrier = pltpu.get_barrier_semaphore()
        for nbr in (left, right):
            pl.semaphore_signal(
                barrier, inc=1, device_id=(nbr,),
                device_id_type=pl.DeviceIdType.MESH,
            )
        pl.semaphore_wait(barrier, 2)

        def cR(k):
            return lax.rem(my + N_DEV - k, N_DEV)

        def cL(k):
            return lax.rem(my + k, N_DEV)

        def send_pair(srcR, dstR, srcL, dstL, step):
            rdR = pltpu.make_async_remote_copy(
                src_ref=srcR, dst_ref=dstR,
                send_sem=ssR.at[step], recv_sem=rsR.at[step],
                device_id=(right,), device_id_type=pl.DeviceIdType.MESH,
            )
            rdL = pltpu.make_async_remote_copy(
                src_ref=srcL, dst_ref=dstL,
                send_sem=ssL.at[step], recv_sem=rsL.at[step],
                device_id=(left,), device_id_type=pl.DeviceIdType.MESH,
            )
            rdR.start()
            rdL.start()
            rdR.wait()
            rdL.wait()

        send_pair(
            p_ref.at[pl.ds(cR(0) * C, C), pl.ds(0, HD)], rbR.at[0],
            p_ref.at[pl.ds(cL(0) * C, C), pl.ds(HD, HD)], rbL.at[0],
            0,
        )
        for step in (1, 2):
            rbR[step - 1, :, :] = (
                rbR[step - 1, :, :] + p_ref[pl.ds(cR(step) * C, C), pl.ds(0, HD)]
            )
            rbL[step - 1, :, :] = (
                rbL[step - 1, :, :] + p_ref[pl.ds(cL(step) * C, C), pl.ds(HD, HD)]
            )
            send_pair(rbR.at[step - 1], rbR.at[step],
                      rbL.at[step - 1], rbL.at[step], step)

        oR = cR(3)
        oL = cL(3)
        agR[pl.ds(oR, 1), :, :] = (
            rbR[2, :, :] + p_ref[pl.ds(oR * C, C), pl.ds(0, HD)]
        )[None]
        agL[pl.ds(oL, 1), :, :] = (
            rbL[2, :, :] + p_ref[pl.ds(oL * C, C), pl.ds(HD, HD)]
        )[None]

        for s_ag in range(3):
            gR = lax.rem(my + 1 + N_DEV - s_ag, N_DEV)
            gL = lax.rem(my + N_DEV - 1 + s_ag, N_DEV)
            send_pair(
                agR.at[pl.ds(gR, 1)], agR.at[pl.ds(gR, 1)],
                agL.at[pl.ds(gL, 1)], agL.at[pl.ds(gL, 1)],
                3 + s_ag,
            )

        for c in range(N_DEV):
            rows = pl.ds(c * C, C)
            o_ref[rows, pl.ds(0, HD)] = (
                r_ref[rows, pl.ds(0, HD)]
                + g_ref[:, pl.ds(0, HD)] * agR[c, :, :].astype(jnp.float32)
            )
            o_ref[rows, pl.ds(HD, HD)] = (
                r_ref[rows, pl.ds(HD, HD)]
                + g_ref[:, pl.ds(HD, HD)] * agL[c, :, :].astype(jnp.float32)
            )

    return pl.pallas_call(
        body,
        in_specs=[
            pl.BlockSpec(memory_space=pltpu.VMEM),
            pl.BlockSpec(memory_space=pltpu.VMEM),
            pl.BlockSpec(memory_space=pltpu.VMEM),
        ],
        out_specs=pl.BlockSpec(memory_space=pltpu.VMEM),
        out_shape=jax.ShapeDtypeStruct((S, D), jnp.float32),
        scratch_shapes=[
            pltpu.VMEM((3, C, HD), jnp.bfloat16),
            pltpu.VMEM((3, C, HD), jnp.bfloat16),
            pltpu.VMEM((N_DEV, C, HD), jnp.bfloat16),
            pltpu.VMEM((N_DEV, C, HD), jnp.bfloat16),
            pltpu.SemaphoreType.DMA((6,)),
            pltpu.SemaphoreType.DMA((6,)),
            pltpu.SemaphoreType.DMA((6,)),
            pltpu.SemaphoreType.DMA((6,)),
        ],
        compiler_params=pltpu.CompilerParams(
            collective_id=collective_id,
            vmem_limit_bytes=100 * 1024 * 1024,
        ),
    )(partial, resid, gate)


def kernel(x, Wq, Wk, Wv, Wo, t_emb, W_mod, W_ff1, W_ff2):
    x2 = x.reshape(S, D)

    mod = t_emb @ W_mod
    sa, sha, ga, sm, shm, gm = jnp.split(mod, 6, axis=-1)

    bf16 = jnp.bfloat16
    Wq, Wk, Wv, Wo = Wq.astype(bf16), Wk.astype(bf16), Wv.astype(bf16), Wo.astype(bf16)
    W_ff1, W_ff2 = W_ff1.astype(bf16), W_ff2.astype(bf16)

    Q, K, V = _ln_mod_matmul3(x2, sa, sha, Wq, Wk, Wv)

    attn = _attention(Q, K, V)

    attn_part = _matmul(attn, Wo)
    x1 = _allreduce_residual(attn_part, x2, ga, collective_id=0)

    ffn_part = _ln_mod_ffn_partial(x1, sm, shm, W_ff1, W_ff2)
    out = _allreduce_residual(ffn_part, x1, gm, collective_id=1)

    return out.reshape(1, S, D)


# device time: 436797 ns/iter; 1.1594x vs baseline; 1.1594x over previous
import jax
import jax.numpy as jnp
from jax import lax
from jax.experimental import pallas as pl
from jax.experimental.pallas import tpu as pltpu

jax.config.update("jax_compilation_cache_dir", "/tmp/jax_comp_cache")
jax.config.update("jax_persistent_cache_min_entry_size_bytes", -1)
jax.config.update("jax_persistent_cache_min_compile_time_secs", 0.0)

N_DEV = 4
S = 4096
D = 1024
H = 8
DH = 128
BLK = 512
EPS = 1e-5
SCALE = 0.08838834764831843


def _ln_mod_matmul3(x2, scale_v, shift_v, Wa, Wb, Wc):

    def body(x_ref, sc_ref, sh_ref, wa_ref, wb_ref, wc_ref, a_ref, b_ref, c_ref):
        xb = x_ref[...]
        m = jnp.mean(xb, axis=1, keepdims=True)
        xc = xb - m
        var = jnp.mean(xc * xc, axis=1, keepdims=True)
        xn = xc * lax.rsqrt(var + EPS)
        xm = (xn * (1.0 + sc_ref[...]) + sh_ref[...]).astype(jnp.bfloat16)
        a_ref[...] = (
            jnp.dot(xm, wa_ref[...], preferred_element_type=jnp.float32) * SCALE
        ).astype(jnp.bfloat16)
        b_ref[...] = jnp.dot(
            xm, wb_ref[...], preferred_element_type=jnp.float32
        ).astype(jnp.bfloat16)
        c_ref[...] = jnp.dot(
            xm, wc_ref[...], preferred_element_type=jnp.float32
        ).astype(jnp.bfloat16)

    vec_spec = pl.BlockSpec((1, D), lambda i: (0, 0))
    w_spec = pl.BlockSpec((D, D), lambda i: (0, 0))
    seq_spec = pl.BlockSpec((BLK, D), lambda i: (i, 0))
    out = jax.ShapeDtypeStruct((S, D), jnp.bfloat16)
    return pl.pallas_call(
        body,
        grid=(S // BLK,),
        in_specs=[seq_spec, vec_spec, vec_spec, w_spec, w_spec, w_spec],
        out_specs=(seq_spec, seq_spec, seq_spec),
        out_shape=(out, out, out),
    )(x2, scale_v, shift_v, Wa, Wb, Wc)


def _attention(Q, K, V):

    def body(q_ref, k_ref, v_ref, o_ref):
        q = q_ref[...]
        k = k_ref[...]
        s = lax.dot_general(
            q, k, (((1,), (1,)), ((), ())), preferred_element_type=jnp.float32
        )
        p = jnp.exp(s)
        l = jnp.sum(p, axis=1, keepdims=True)
        o = jnp.dot(
            p.astype(jnp.bfloat16), v_ref[...], preferred_element_type=jnp.float32
        )
        o_ref[...] = (o / l).astype(jnp.bfloat16)

    q_spec = pl.BlockSpec((BLK, DH), lambda h, qb: (qb, h))
    kv_spec = pl.BlockSpec((S, DH), lambda h, qb: (0, h))
    return pl.pallas_call(
        body,
        grid=(H, S // BLK),
        in_specs=[q_spec, kv_spec, kv_spec],
        out_specs=q_spec,
        out_shape=jax.ShapeDtypeStruct((S, H * DH), jnp.bfloat16),
    )(Q, K, V)


def _matmul(A, B):

    def body(a_ref, b_ref, o_ref):
        o_ref[...] = jnp.dot(
            a_ref[...], b_ref[...], preferred_element_type=jnp.float32
        ).astype(jnp.bfloat16)

    return pl.pallas_call(
        body,
        grid=(S // BLK,),
        in_specs=[
            pl.BlockSpec((BLK, D), lambda i: (i, 0)),
            pl.BlockSpec((D, D), lambda i: (0, 0)),
        ],
        out_specs=pl.BlockSpec((BLK, D), lambda i: (i, 0)),
        out_shape=jax.ShapeDtypeStruct((S, D), jnp.bfloat16),
    )(A, B)


def _ln_mod_ffn_partial(x2, scale_v, shift_v, W1, W2):

    def body(x_ref, sc_ref, sh_ref, w1_ref, w2_ref, o_ref):
        xb = x_ref[...]
        m = jnp.mean(xb, axis=1, keepdims=True)
        xc = xb - m
        var = jnp.mean(xc * xc, axis=1, keepdims=True)
        xn = xc * lax.rsqrt(var + EPS)
        xm = (xn * (1.0 + sc_ref[...]) + sh_ref[...]).astype(jnp.bfloat16)
        h = jnp.dot(xm, w1_ref[...], preferred_element_type=jnp.float32)
        h = (h * jax.nn.sigmoid(h)).astype(jnp.bfloat16)
        o_ref[...] = jnp.dot(
            h, w2_ref[...], preferred_element_type=jnp.float32
        ).astype(jnp.bfloat16)

    vec_spec = pl.BlockSpec((1, D), lambda i: (0, 0))
    w_spec = pl.BlockSpec((D, D), lambda i: (0, 0))
    seq_spec = pl.BlockSpec((BLK, D), lambda i: (i, 0))
    return pl.pallas_call(
        body,
        grid=(S // BLK,),
        in_specs=[seq_spec, vec_spec, vec_spec, w_spec, w_spec],
        out_specs=seq_spec,
        out_shape=jax.ShapeDtypeStruct((S, D), jnp.bfloat16),
    )(x2, scale_v, shift_v, W1, W2)


def _fused_attn_ar(Q, K, V, Wo):

    CH = S // (2 * N_DEV)

    def attn_rows(q_ref, k_ref, v_ref, wo_ref, row0, abuf):

        def head_body(h, carry):
            hc = pl.ds(h * DH, DH)
            q = q_ref[pl.ds(row0, CH), hc]
            k = k_ref[:, hc]
            s = lax.dot_general(
                q, k, (((1,), (1,)), ((), ())),
                preferred_element_type=jnp.float32,
            )
            p = jnp.exp(s)
            l = jnp.sum(p, axis=1, keepdims=True)
            o = jnp.dot(
                p.astype(jnp.bfloat16), v_ref[:, hc],
                preferred_element_type=jnp.float32,
            )
            abuf[:, hc] = (o / l).astype(jnp.bfloat16)
            return carry

        lax.fori_loop(0, H, head_body, 0)
        return jnp.dot(
            abuf[...], wo_ref[...], preferred_element_type=jnp.float32
        ).astype(jnp.bfloat16)

    def body(q_ref, k_ref, v_ref, wo_ref, o_ref,
             abR, abL, pbR, pbL, rbR, rbL, ssR, rsR, ssL, rsL):
        my = lax.axis_index("i")
        left = lax.rem(my + N_DEV - 1, N_DEV)
        right = lax.rem(my + 1, N_DEV)

        barrier = pltpu.get_barrier_semaphore()
        for nbr in (left, right):
            pl.semaphore_signal(
                barrier, inc=1, device_id=(nbr,),
                device_id_type=pl.DeviceIdType.MESH,
            )
        pl.semaphore_wait(barrier, 2)

        def cR(k):
            return lax.rem(my + N_DEV - k, N_DEV)

        def cL(k):
            return lax.rem(my + k, N_DEV)

        def mk_pair(srcR, dstR, srcL, dstL, step):
            rdR = pltpu.make_async_remote_copy(
                src_ref=srcR, dst_ref=dstR,
                send_sem=ssR.at[step], recv_sem=rsR.at[step],
                device_id=(right,), device_id_type=pl.DeviceIdType.MESH,
            )
            rdL = pltpu.make_async_remote_copy(
                src_ref=srcL, dst_ref=dstL,
                send_sem=ssL.at[step], recv_sem=rsL.at[step],
                device_id=(left,), device_id_type=pl.DeviceIdType.MESH,
            )
            rdR.start()
            rdL.start()
            return rdR, rdL

        descs = []
        pbR[0, :, :] = attn_rows(q_ref, k_ref, v_ref, wo_ref, cR(0) * CH, abR)
        pbL[0, :, :] = attn_rows(
            q_ref, k_ref, v_ref, wo_ref, S // 2 + cL(0) * CH, abL
        )
        descs.append(mk_pair(pbR.at[0], rbR.at[0], pbL.at[0], rbL.at[0], 0))

        for k in (1, 2, 3):
            pbR[1, :, :] = attn_rows(q_ref, k_ref, v_ref, wo_ref, cR(k) * CH, abR)
            pbL[1, :, :] = attn_rows(
                q_ref, k_ref, v_ref, wo_ref, S // 2 + cL(k) * CH, abL
            )
            rdR, rdL = descs[k - 1]
            rdR.wait_recv()
            rdL.wait_recv()
            if k < 3:
                rbR[k - 1, :, :] = rbR[k - 1, :, :] + pbR[1, :, :]
                rbL[k - 1, :, :] = rbL[k - 1, :, :] + pbL[1, :, :]
                descs.append(
                    mk_pair(rbR.at[k - 1], rbR.at[k], rbL.at[k - 1], rbL.at[k], k)
                )

        oR = cR(3)
        oL = cL(3)
        o_ref[pl.ds(oR * CH, CH), :] = rbR[2, :, :] + pbR[1, :, :]
        o_ref[pl.ds(S // 2 + oL * CH, CH), :] = rbL[2, :, :] + pbL[1, :, :]

        for rdR, rdL in descs:
            rdR.wait_send()
            rdL.wait_send()

        for s_ag in range(3):
            gR = lax.rem(my + 1 + N_DEV - s_ag, N_DEV)
            gL = lax.rem(my + N_DEV - 1 + s_ag, N_DEV)
            rdR, rdL = mk_pair(
                o_ref.at[pl.ds(gR * CH, CH), :],
                o_ref.at[pl.ds(gR * CH, CH), :],
                o_ref.at[pl.ds(S // 2 + gL * CH, CH), :],
                o_ref.at[pl.ds(S // 2 + gL * CH, CH), :],
                3 + s_ag,
            )
            rdR.wait()
            rdL.wait()

    return pl.pallas_call(
        body,
        in_specs=[pl.BlockSpec(memory_space=pltpu.VMEM)] * 4,
        out_specs=pl.BlockSpec(memory_space=pltpu.VMEM),
        out_shape=jax.ShapeDtypeStruct((S, D), jnp.bfloat16),
        scratch_shapes=[
            pltpu.VMEM((CH, D), jnp.bfloat16),
            pltpu.VMEM((CH, D), jnp.bfloat16),
            pltpu.VMEM((2, CH, D), jnp.bfloat16),
            pltpu.VMEM((2, CH, D), jnp.bfloat16),
            pltpu.VMEM((3, CH, D), jnp.bfloat16),
            pltpu.VMEM((3, CH, D), jnp.bfloat16),
            pltpu.SemaphoreType.DMA((6,)),
            pltpu.SemaphoreType.DMA((6,)),
            pltpu.SemaphoreType.DMA((6,)),
            pltpu.SemaphoreType.DMA((6,)),
        ],
        compiler_params=pltpu.CompilerParams(
            collective_id=0,
            vmem_limit_bytes=63 * 1024 * 1024,
        ),
    )(Q, K, V, Wo)


def _gated_residual(resid, gate, ar):

    def body(r_ref, g_ref, a_ref, o_ref):
        o_ref[...] = r_ref[...] + g_ref[...] * a_ref[...].astype(jnp.float32)

    return pl.pallas_call(
        body,
        grid=(S // BLK,),
        in_specs=[
            pl.BlockSpec((BLK, D), lambda i: (i, 0)),
            pl.BlockSpec((1, D), lambda i: (0, 0)),
            pl.BlockSpec((BLK, D), lambda i: (i, 0)),
        ],
        out_specs=pl.BlockSpec((BLK, D), lambda i: (i, 0)),
        out_shape=jax.ShapeDtypeStruct((S, D), jnp.float32),
    )(resid, gate, ar)


def _allreduce_residual(partial, resid, gate, collective_id):

    C = S // N_DEV
    HD = D // 2

    def body(p_ref, r_ref, g_ref, o_ref, rbR, rbL, agR, agL, ssR, rsR, ssL, rsL):
        my = lax.axis_index("i")
        left = lax.rem(my + N_DEV - 1, N_DEV)
        right = lax.rem(my + 1, N_DEV)

        barrier = pltpu.get_barrier_semaphore()
        for nbr in (left, right):
            pl.semaphore_signal(
                barrier, inc=1, device_id=(nbr,),
                device_id_type=pl.DeviceIdType.MESH,
            )
        pl.semaphore_wait(barrier, 2)

        def cR(k):
            return lax.rem(my + N_DEV - k, N_DEV)

        def cL(k):
            return lax.rem(my + k, N_DEV)

        def send_pair(srcR, dstR, srcL, dstL, step):
            rdR = pltpu.make_async_remote_copy(
                src_ref=srcR, dst_ref=dstR,
                send_sem=ssR.at[step], recv_sem=rsR.at[step],
                device_id=(right,), device_id_type=pl.DeviceIdType.MESH,
            )
            rdL = pltpu.make_async_remote_copy(
                src_ref=srcL, dst_ref=dstL,
                send_sem=ssL.at[step], recv_sem=rsL.at[step],
                device_id=(left,), device_id_type=pl.DeviceIdType.MESH,
            )
            rdR.start()
            rdL.start()
            rdR.wait()
            rdL.wait()

        send_pair(
            p_ref.at[pl.ds(cR(0) * C, C), pl.ds(0, HD)], rbR.at[0],
            p_ref.at[pl.ds(cL(0) * C, C), pl.ds(HD, HD)], rbL.at[0],
            0,
        )
        for step in (1, 2):
            rbR[step - 1, :, :] = (
                rbR[step - 1, :, :] + p_ref[pl.ds(cR(step) * C, C), pl.ds(0, HD)]
            )
            rbL[step - 1, :, :] = (
                rbL[step - 1, :, :] + p_ref[pl.ds(cL(step) * C, C), pl.ds(HD, HD)]
            )
            send_pair(rbR.at[step - 1], rbR.at[step],
                      rbL.at[step - 1], rbL.at[step], step)

        oR = cR(3)
        oL = cL(3)
        agR[pl.ds(oR, 1), :, :] = (
            rbR[2, :, :] + p_ref[pl.ds(oR * C, C), pl.ds(0, HD)]
        )[None]
        agL[pl.ds(oL, 1), :, :] = (
            rbL[2, :, :] + p_ref[pl.ds(oL * C, C), pl.ds(HD, HD)]
        )[None]

        for s_ag in range(3):
            gR = lax.rem(my + 1 + N_DEV - s_ag, N_DEV)
            gL = lax.rem(my + N_DEV - 1 + s_ag, N_DEV)
            send_pair(
                agR.at[pl.ds(gR, 1)], agR.at[pl.ds(gR, 1)],
                agL.at[pl.ds(gL, 1)], agL.at[pl.ds(gL, 1)],
                3 + s_ag,
            )

        for c in range(N_DEV):
            rows = pl.ds(c * C, C)
            o_ref[rows, pl.ds(0, HD)] = (
                r_ref[rows, pl.ds(0, HD)]
                + g_ref[:, pl.ds(0, HD)] * agR[c, :, :].astype(jnp.float32)
            )
            o_ref[rows, pl.ds(HD, HD)] = (
                r_ref[rows, pl.ds(HD, HD)]
                + g_ref[:, pl.ds(HD, HD)] * agL[c, :, :].astype(jnp.float32)
            )

    return pl.pallas_call(
        body,
        in_specs=[
            pl.BlockSpec(memory_space=pltpu.VMEM),
            pl.BlockSpec(memory_space=pltpu.VMEM),
            pl.BlockSpec(memory_space=pltpu.VMEM),
        ],
        out_specs=pl.BlockSpec(memory_space=pltpu.VMEM),
        out_shape=jax.ShapeDtypeStruct((S, D), jnp.float32),
        scratch_shapes=[
            pltpu.VMEM((3, C, HD), jnp.bfloat16),
            pltpu.VMEM((3, C, HD), jnp.bfloat16),
            pltpu.VMEM((N_DEV, C, HD), jnp.bfloat16),
            pltpu.VMEM((N_DEV, C, HD), jnp.bfloat16),
            pltpu.SemaphoreType.DMA((6,)),
            pltpu.SemaphoreType.DMA((6,)),
            pltpu.SemaphoreType.DMA((6,)),
            pltpu.SemaphoreType.DMA((6,)),
        ],
        compiler_params=pltpu.CompilerParams(
            collective_id=collective_id,
            vmem_limit_bytes=100 * 1024 * 1024,
        ),
    )(partial, resid, gate)


def kernel(x, Wq, Wk, Wv, Wo, t_emb, W_mod, W_ff1, W_ff2):
    x2 = x.reshape(S, D)

    mod = t_emb @ W_mod
    sa, sha, ga, sm, shm, gm = jnp.split(mod, 6, axis=-1)

    bf16 = jnp.bfloat16
    Wq, Wk, Wv, Wo = Wq.astype(bf16), Wk.astype(bf16), Wv.astype(bf16), Wo.astype(bf16)
    W_ff1, W_ff2 = W_ff1.astype(bf16), W_ff2.astype(bf16)

    Q, K, V = _ln_mod_matmul3(x2, sa, sha, Wq, Wk, Wv)

    ar1 = _fused_attn_ar(Q, K, V, Wo)
    x1 = _gated_residual(x2, ga, ar1)

    ffn_part = _ln_mod_ffn_partial(x1, sm, shm, W_ff1, W_ff2)
    out = _allreduce_residual(ffn_part, x1, gm, collective_id=1)

    return out.reshape(1, S, D)


# device time: 421804 ns/iter; 1.2006x vs baseline; 1.0355x over previous
import jax
import jax.numpy as jnp
from jax import lax
from jax.experimental import pallas as pl
from jax.experimental.pallas import tpu as pltpu

jax.config.update("jax_compilation_cache_dir", "/tmp/jax_comp_cache")
jax.config.update("jax_persistent_cache_min_entry_size_bytes", -1)
jax.config.update("jax_persistent_cache_min_compile_time_secs", 0.0)

N_DEV = 4
S = 4096
D = 1024
H = 8
DH = 128
BLK = 512
EPS = 1e-5
SCALE = 0.08838834764831843


def _ln_mod_matmul3(x2, scale_v, shift_v, Wa, Wb, Wc):

    def body(x_ref, sc_ref, sh_ref, wa_ref, wb_ref, wc_ref, a_ref, b_ref, c_ref):
        xb = x_ref[...]
        m = jnp.mean(xb, axis=1, keepdims=True)
        xc = xb - m
        var = jnp.mean(xc * xc, axis=1, keepdims=True)
        xn = xc * lax.rsqrt(var + EPS)
        xm = (xn * (1.0 + sc_ref[...]) + sh_ref[...]).astype(jnp.bfloat16)
        a_ref[...] = (
            jnp.dot(xm, wa_ref[...], preferred_element_type=jnp.float32) * SCALE
        ).astype(jnp.bfloat16)
        b_ref[...] = jnp.dot(
            xm, wb_ref[...], preferred_element_type=jnp.float32
        ).astype(jnp.bfloat16)
        c_ref[...] = jnp.dot(
            xm, wc_ref[...], preferred_element_type=jnp.float32
        ).astype(jnp.bfloat16)

    vec_spec = pl.BlockSpec((1, D), lambda i: (0, 0))
    w_spec = pl.BlockSpec((D, D), lambda i: (0, 0))
    seq_spec = pl.BlockSpec((BLK, D), lambda i: (i, 0))
    out = jax.ShapeDtypeStruct((S, D), jnp.bfloat16)
    return pl.pallas_call(
        body,
        grid=(S // BLK,),
        in_specs=[seq_spec, vec_spec, vec_spec, w_spec, w_spec, w_spec],
        out_specs=(seq_spec, seq_spec, seq_spec),
        out_shape=(out, out, out),
    )(x2, scale_v, shift_v, Wa, Wb, Wc)


def _attention(Q, K, V):

    def body(q_ref, k_ref, v_ref, o_ref):
        q = q_ref[...]
        k = k_ref[...]
        s = lax.dot_general(
            q, k, (((1,), (1,)), ((), ())), preferred_element_type=jnp.float32
        )
        p = jnp.exp(s)
        l = jnp.sum(p, axis=1, keepdims=True)
        o = jnp.dot(
            p.astype(jnp.bfloat16), v_ref[...], preferred_element_type=jnp.float32
        )
        o_ref[...] = (o / l).astype(jnp.bfloat16)

    q_spec = pl.BlockSpec((BLK, DH), lambda h, qb: (qb, h))
    kv_spec = pl.BlockSpec((S, DH), lambda h, qb: (0, h))
    return pl.pallas_call(
        body,
        grid=(H, S // BLK),
        in_specs=[q_spec, kv_spec, kv_spec],
        out_specs=q_spec,
        out_shape=jax.ShapeDtypeStruct((S, H * DH), jnp.bfloat16),
    )(Q, K, V)


def _matmul(A, B):

    def body(a_ref, b_ref, o_ref):
        o_ref[...] = jnp.dot(
            a_ref[...], b_ref[...], preferred_element_type=jnp.float32
        ).astype(jnp.bfloat16)

    return pl.pallas_call(
        body,
        grid=(S // BLK,),
        in_specs=[
            pl.BlockSpec((BLK, D), lambda i: (i, 0)),
            pl.BlockSpec((D, D), lambda i: (0, 0)),
        ],
        out_specs=pl.BlockSpec((BLK, D), lambda i: (i, 0)),
        out_shape=jax.ShapeDtypeStruct((S, D), jnp.bfloat16),
    )(A, B)


def _ln_mod_ffn_partial(x2, scale_v, shift_v, W1, W2):

    def body(x_ref, sc_ref, sh_ref, w1_ref, w2_ref, o_ref):
        xb = x_ref[...]
        m = jnp.mean(xb, axis=1, keepdims=True)
        xc = xb - m
        var = jnp.mean(xc * xc, axis=1, keepdims=True)
        xn = xc * lax.rsqrt(var + EPS)
        xm = (xn * (1.0 + sc_ref[...]) + sh_ref[...]).astype(jnp.bfloat16)
        h = jnp.dot(xm, w1_ref[...], preferred_element_type=jnp.float32)
        h = (h * jax.nn.sigmoid(h)).astype(jnp.bfloat16)
        o_ref[...] = jnp.dot(
            h, w2_ref[...], preferred_element_type=jnp.float32
        ).astype(jnp.bfloat16)

    vec_spec = pl.BlockSpec((1, D), lambda i: (0, 0))
    w_spec = pl.BlockSpec((D, D), lambda i: (0, 0))
    seq_spec = pl.BlockSpec((BLK, D), lambda i: (i, 0))
    return pl.pallas_call(
        body,
        grid=(S // BLK,),
        in_specs=[seq_spec, vec_spec, vec_spec, w_spec, w_spec],
        out_specs=seq_spec,
        out_shape=jax.ShapeDtypeStruct((S, D), jnp.bfloat16),
    )(x2, scale_v, shift_v, W1, W2)


def _fused_attn_ar(Q, K, V, Wo):

    CH = S // (2 * N_DEV)

    def attn_rows(q_ref, k_ref, v_ref, wo_ref, row0, abuf):

        def head_body(h, carry):
            hc = pl.ds(h * DH, DH)
            q = q_ref[pl.ds(row0, CH), hc]
            k = k_ref[:, hc]
            s = lax.dot_general(
                q, k, (((1,), (1,)), ((), ())),
                preferred_element_type=jnp.float32,
            )
            p = jnp.exp(s)
            l = jnp.sum(p, axis=1, keepdims=True)
            o = jnp.dot(
                p.astype(jnp.bfloat16), v_ref[:, hc],
                preferred_element_type=jnp.float32,
            )
            abuf[:, hc] = (o / l).astype(jnp.bfloat16)
            return carry

        lax.fori_loop(0, H, head_body, 0)
        return jnp.dot(
            abuf[...], wo_ref[...], preferred_element_type=jnp.float32
        ).astype(jnp.bfloat16)

    def body(q_ref, k_ref, v_ref, wo_ref, o_ref,
             abR, abL, pbR, pbL, rbR, rbL, ssR, rsR, ssL, rsL):
        my = lax.axis_index("i")
        left = lax.rem(my + N_DEV - 1, N_DEV)
        right = lax.rem(my + 1, N_DEV)

        barrier = pltpu.get_barrier_semaphore()
        for nbr in (left, right):
            pl.semaphore_signal(
                barrier, inc=1, device_id=(nbr,),
                device_id_type=pl.DeviceIdType.MESH,
            )
        pl.semaphore_wait(barrier, 2)

        def cR(k):
            return lax.rem(my + N_DEV - k, N_DEV)

        def cL(k):
            return lax.rem(my + k, N_DEV)

        def mk_pair(srcR, dstR, srcL, dstL, step):
            rdR = pltpu.make_async_remote_copy(
                src_ref=srcR, dst_ref=dstR,
                send_sem=ssR.at[step], recv_sem=rsR.at[step],
                device_id=(right,), device_id_type=pl.DeviceIdType.MESH,
            )
            rdL = pltpu.make_async_remote_copy(
                src_ref=srcL, dst_ref=dstL,
                send_sem=ssL.at[step], recv_sem=rsL.at[step],
                device_id=(left,), device_id_type=pl.DeviceIdType.MESH,
            )
            rdR.start()
            rdL.start()
            return rdR, rdL

        descs = []
        pbR[0, :, :] = attn_rows(q_ref, k_ref, v_ref, wo_ref, cR(0) * CH, abR)
        pbL[0, :, :] = attn_rows(
            q_ref, k_ref, v_ref, wo_ref, S // 2 + cL(0) * CH, abL
        )
        descs.append(mk_pair(pbR.at[0], rbR.at[0], pbL.at[0], rbL.at[0], 0))

        for k in (1, 2, 3):
            pbR[1, :, :] = attn_rows(q_ref, k_ref, v_ref, wo_ref, cR(k) * CH, abR)
            pbL[1, :, :] = attn_rows(
                q_ref, k_ref, v_ref, wo_ref, S // 2 + cL(k) * CH, abL
            )
            rdR, rdL = descs[k - 1]
            rdR.wait_recv()
            rdL.wait_recv()
            if k < 3:
                rbR[k - 1, :, :] = rbR[k - 1, :, :] + pbR[1, :, :]
                rbL[k - 1, :, :] = rbL[k - 1, :, :] + pbL[1, :, :]
                descs.append(
                    mk_pair(rbR.at[k - 1], rbR.at[k], rbL.at[k - 1], rbL.at[k], k)
                )

        oR = cR(3)
        oL = cL(3)
        o_ref[pl.ds(oR * CH, CH), :] = rbR[2, :, :] + pbR[1, :, :]
        o_ref[pl.ds(S // 2 + oL * CH, CH), :] = rbL[2, :, :] + pbL[1, :, :]

        for rdR, rdL in descs:
            rdR.wait_send()
            rdL.wait_send()

        for s_ag in range(3):
            gR = lax.rem(my + 1 + N_DEV - s_ag, N_DEV)
            gL = lax.rem(my + N_DEV - 1 + s_ag, N_DEV)
            rdR, rdL = mk_pair(
                o_ref.at[pl.ds(gR * CH, CH), :],
                o_ref.at[pl.ds(gR * CH, CH), :],
                o_ref.at[pl.ds(S // 2 + gL * CH, CH), :],
                o_ref.at[pl.ds(S // 2 + gL * CH, CH), :],
                3 + s_ag,
            )
            rdR.wait()
            rdL.wait()

    return pl.pallas_call(
        body,
        in_specs=[pl.BlockSpec(memory_space=pltpu.VMEM)] * 4,
        out_specs=pl.BlockSpec(memory_space=pltpu.VMEM),
        out_shape=jax.ShapeDtypeStruct((S, D), jnp.bfloat16),
        scratch_shapes=[
            pltpu.VMEM((CH, D), jnp.bfloat16),
            pltpu.VMEM((CH, D), jnp.bfloat16),
            pltpu.VMEM((2, CH, D), jnp.bfloat16),
            pltpu.VMEM((2, CH, D), jnp.bfloat16),
            pltpu.VMEM((3, CH, D), jnp.bfloat16),
            pltpu.VMEM((3, CH, D), jnp.bfloat16),
            pltpu.SemaphoreType.DMA((6,)),
            pltpu.SemaphoreType.DMA((6,)),
            pltpu.SemaphoreType.DMA((6,)),
            pltpu.SemaphoreType.DMA((6,)),
        ],
        compiler_params=pltpu.CompilerParams(
            collective_id=0,
            vmem_limit_bytes=63 * 1024 * 1024,
        ),
    )(Q, K, V, Wo)


def _gated_residual(resid, gate, ar):

    def body(r_ref, g_ref, a_ref, o_ref):
        o_ref[...] = r_ref[...] + g_ref[...] * a_ref[...].astype(jnp.float32)

    return pl.pallas_call(
        body,
        grid=(S // BLK,),
        in_specs=[
            pl.BlockSpec((BLK, D), lambda i: (i, 0)),
            pl.BlockSpec((1, D), lambda i: (0, 0)),
            pl.BlockSpec((BLK, D), lambda i: (i, 0)),
        ],
        out_specs=pl.BlockSpec((BLK, D), lambda i: (i, 0)),
        out_shape=jax.ShapeDtypeStruct((S, D), jnp.float32),
    )(resid, gate, ar)


def _fused_ffn_ar(x1, scale_v, shift_v, gate, W1, W2):

    CH = S // (2 * N_DEV)

    def body(x_ref, sc_ref, sh_ref, g_ref, w1_ref, w2_ref, o_ref,
             pbR, pbL, rbR, rbL, agR, agL, ssR, rsR, ssL, rsL):
        my = lax.axis_index("i")
        left = lax.rem(my + N_DEV - 1, N_DEV)
        right = lax.rem(my + 1, N_DEV)

        barrier = pltpu.get_barrier_semaphore()
        for nbr in (left, right):
            pl.semaphore_signal(
                barrier, inc=1, device_id=(nbr,),
                device_id_type=pl.DeviceIdType.MESH,
            )
        pl.semaphore_wait(barrier, 2)

        def cR(k):
            return lax.rem(my + N_DEV - k, N_DEV)

        def cL(k):
            return lax.rem(my + k, N_DEV)

        def mk_pair(srcR, dstR, srcL, dstL, step):
            rdR = pltpu.make_async_remote_copy(
                src_ref=srcR, dst_ref=dstR,
                send_sem=ssR.at[step], recv_sem=rsR.at[step],
                device_id=(right,), device_id_type=pl.DeviceIdType.MESH,
            )
            rdL = pltpu.make_async_remote_copy(
                src_ref=srcL, dst_ref=dstL,
                send_sem=ssL.at[step], recv_sem=rsL.at[step],
                device_id=(left,), device_id_type=pl.DeviceIdType.MESH,
            )
            rdR.start()
            rdL.start()
            return rdR, rdL

        def ffn_rows(row0):
            xb = x_ref[pl.ds(row0, CH), :]
            m = jnp.mean(xb, axis=1, keepdims=True)
            xc = xb - m
            var = jnp.mean(xc * xc, axis=1, keepdims=True)
            xn = xc * lax.rsqrt(var + EPS)
            xm = (xn * (1.0 + sc_ref[...]) + sh_ref[...]).astype(jnp.bfloat16)
            h = jnp.dot(xm, w1_ref[...], preferred_element_type=jnp.float32)
            h = (h * jax.nn.sigmoid(h)).astype(jnp.bfloat16)
            return jnp.dot(
                h, w2_ref[...], preferred_element_type=jnp.float32
            ).astype(jnp.bfloat16)

        descs = []
        pbR[0, :, :] = ffn_rows(cR(0) * CH)
        pbL[0, :, :] = ffn_rows(S // 2 + cL(0) * CH)
        descs.append(mk_pair(pbR.at[0], rbR.at[0], pbL.at[0], rbL.at[0], 0))

        for k in (1, 2, 3):
            pbR[1, :, :] = ffn_rows(cR(k) * CH)
            pbL[1, :, :] = ffn_rows(S // 2 + cL(k) * CH)
            rdR, rdL = descs[k - 1]
            rdR.wait_recv()
            rdL.wait_recv()
            if k < 3:
                rbR[k - 1, :, :] = rbR[k - 1, :, :] + pbR[1, :, :]
                rbL[k - 1, :, :] = rbL[k - 1, :, :] + pbL[1, :, :]
                descs.append(
                    mk_pair(rbR.at[k - 1], rbR.at[k], rbL.at[k - 1], rbL.at[k], k)
                )

        oR = cR(3)
        oL = cL(3)
        agR[pl.ds(oR, 1), :, :] = (rbR[2, :, :] + pbR[1, :, :])[None]
        agL[pl.ds(oL, 1), :, :] = (rbL[2, :, :] + pbL[1, :, :])[None]

        for rdR, rdL in descs:
            rdR.wait_send()
            rdL.wait_send()

        for s_ag in range(3):
            gR = lax.rem(my + 1 + N_DEV - s_ag, N_DEV)
            gL = lax.rem(my + N_DEV - 1 + s_ag, N_DEV)
            rdR, rdL = mk_pair(
                agR.at[pl.ds(gR, 1)], agR.at[pl.ds(gR, 1)],
                agL.at[pl.ds(gL, 1)], agL.at[pl.ds(gL, 1)],
                3 + s_ag,
            )
            rdR.wait()
            rdL.wait()

        for c in range(N_DEV):
            rowsR = pl.ds(c * CH, CH)
            rowsL = pl.ds(S // 2 + c * CH, CH)
            o_ref[rowsR, :] = (
                x_ref[rowsR, :] + g_ref[...] * agR[c, :, :].astype(jnp.float32)
            )
            o_ref[rowsL, :] = (
                x_ref[rowsL, :] + g_ref[...] * agL[c, :, :].astype(jnp.float32)
            )

    return pl.pallas_call(
        body,
        in_specs=[pl.BlockSpec(memory_space=pltpu.VMEM)] * 6,
        out_specs=pl.BlockSpec(memory_space=pltpu.VMEM),
        out_shape=jax.ShapeDtypeStruct((S, D), jnp.float32),
        scratch_shapes=[
            pltpu.VMEM((2, CH, D), jnp.bfloat16),
            pltpu.VMEM((2, CH, D), jnp.bfloat16),
            pltpu.VMEM((3, CH, D), jnp.bfloat16),
            pltpu.VMEM((3, CH, D), jnp.bfloat16),
            pltpu.VMEM((N_DEV, CH, D), jnp.bfloat16),
            pltpu.VMEM((N_DEV, CH, D), jnp.bfloat16),
            pltpu.SemaphoreType.DMA((6,)),
            pltpu.SemaphoreType.DMA((6,)),
            pltpu.SemaphoreType.DMA((6,)),
            pltpu.SemaphoreType.DMA((6,)),
        ],
        compiler_params=pltpu.CompilerParams(
            collective_id=1,
            vmem_limit_bytes=63 * 1024 * 1024,
        ),
    )(x1, scale_v, shift_v, gate, W1, W2)


def _allreduce_residual(partial, resid, gate, collective_id):

    C = S // N_DEV
    HD = D // 2

    def body(p_ref, r_ref, g_ref, o_ref, rbR, rbL, agR, agL, ssR, rsR, ssL, rsL):
        my = lax.axis_index("i")
        left = lax.rem(my + N_DEV - 1, N_DEV)
        right = lax.rem(my + 1, N_DEV)

        barrier = pltpu.get_barrier_semaphore()
        for nbr in (left, right):
            pl.semaphore_signal(
                barrier, inc=1, device_id=(nbr,),
                device_id_type=pl.DeviceIdType.MESH,
            )
        pl.semaphore_wait(barrier, 2)

        def cR(k):
            return lax.rem(my + N_DEV - k, N_DEV)

        def cL(k):
            return lax.rem(my + k, N_DEV)

        def send_pair(srcR, dstR, srcL, dstL, step):
            rdR = pltpu.make_async_remote_copy(
                src_ref=srcR, dst_ref=dstR,
                send_sem=ssR.at[step], recv_sem=rsR.at[step],
                device_id=(right,), device_id_type=pl.DeviceIdType.MESH,
            )
            rdL = pltpu.make_async_remote_copy(
                src_ref=srcL, dst_ref=dstL,
                send_sem=ssL.at[step], recv_sem=rsL.at[step],
                device_id=(left,), device_id_type=pl.DeviceIdType.MESH,
            )
            rdR.start()
            rdL.start()
            rdR.wait()
            rdL.wait()

        send_pair(
            p_ref.at[pl.ds(cR(0) * C, C), pl.ds(0, HD)], rbR.at[0],
            p_ref.at[pl.ds(cL(0) * C, C), pl.ds(HD, HD)], rbL.at[0],
            0,
        )
        for step in (1, 2):
            rbR[step - 1, :, :] = (
                rbR[step - 1, :, :] + p_ref[pl.ds(cR(step) * C, C), pl.ds(0, HD)]
            )
            rbL[step - 1, :, :] = (
                rbL[step - 1, :, :] + p_ref[pl.ds(cL(step) * C, C), pl.ds(HD, HD)]
            )
            send_pair(rbR.at[step - 1], rbR.at[step],
                      rbL.at[step - 1], rbL.at[step], step)

        oR = cR(3)
        oL = cL(3)
        agR[pl.ds(oR, 1), :, :] = (
            rbR[2, :, :] + p_ref[pl.ds(oR * C, C), pl.ds(0, HD)]
        )[None]
        agL[pl.ds(oL, 1), :, :] = (
            rbL[2, :, :] + p_ref[pl.ds(oL * C, C), pl.ds(HD, HD)]
        )[None]

        for s_ag in range(3):
            gR = lax.rem(my + 1 + N_DEV - s_ag, N_DEV)
            gL = lax.rem(my + N_DEV - 1 + s_ag, N_DEV)
            send_pair(
                agR.at[pl.ds(gR, 1)], agR.at[pl.ds(gR, 1)],
                agL.at[pl.ds(gL, 1)], agL.at[pl.ds(gL, 1)],
                3 + s_ag,
            )

        for c in range(N_DEV):
            rows = pl.ds(c * C, C)
            o_ref[rows, pl.ds(0, HD)] = (
                r_ref[rows, pl.ds(0, HD)]
                + g_ref[:, pl.ds(0, HD)] * agR[c, :, :].astype(jnp.float32)
            )
            o_ref[rows, pl.ds(HD, HD)] = (
                r_ref[rows, pl.ds(HD, HD)]
                + g_ref[:, pl.ds(HD, HD)] * agL[c, :, :].astype(jnp.float32)
            )

    return pl.pallas_call(
        body,
        in_specs=[
            pl.BlockSpec(memory_space=pltpu.VMEM),
            pl.BlockSpec(memory_space=pltpu.VMEM),
            pl.BlockSpec(memory_space=pltpu.VMEM),
        ],
        out_specs=pl.BlockSpec(memory_space=pltpu.VMEM),
        out_shape=jax.ShapeDtypeStruct((S, D), jnp.float32),
        scratch_shapes=[
            pltpu.VMEM((3, C, HD), jnp.bfloat16),
            pltpu.VMEM((3, C, HD), jnp.bfloat16),
            pltpu.VMEM((N_DEV, C, HD), jnp.bfloat16),
            pltpu.VMEM((N_DEV, C, HD), jnp.bfloat16),
            pltpu.SemaphoreType.DMA((6,)),
            pltpu.SemaphoreType.DMA((6,)),
            pltpu.SemaphoreType.DMA((6,)),
            pltpu.SemaphoreType.DMA((6,)),
        ],
        compiler_params=pltpu.CompilerParams(
            collective_id=collective_id,
            vmem_limit_bytes=100 * 1024 * 1024,
        ),
    )(partial, resid, gate)


def kernel(x, Wq, Wk, Wv, Wo, t_emb, W_mod, W_ff1, W_ff2):
    x2 = x.reshape(S, D)

    mod = t_emb @ W_mod
    sa, sha, ga, sm, shm, gm = jnp.split(mod, 6, axis=-1)

    bf16 = jnp.bfloat16
    Wq, Wk, Wv, Wo = Wq.astype(bf16), Wk.astype(bf16), Wv.astype(bf16), Wo.astype(bf16)
    W_ff1, W_ff2 = W_ff1.astype(bf16), W_ff2.astype(bf16)

    Q, K, V = _ln_mod_matmul3(x2, sa, sha, Wq, Wk, Wv)

    ar1 = _fused_attn_ar(Q, K, V, Wo)
    x1 = _gated_residual(x2, ga, ar1)

    out = _fused_ffn_ar(x1, sm, shm, gm, W_ff1, W_ff2)

    return out.reshape(1, S, D)


# device time: 421734 ns/iter; 1.2008x vs baseline; 1.0002x over previous
import jax
import jax.numpy as jnp
from jax import lax
from jax.experimental import pallas as pl
from jax.experimental.pallas import tpu as pltpu

jax.config.update("jax_compilation_cache_dir", "/tmp/jax_comp_cache")
jax.config.update("jax_persistent_cache_min_entry_size_bytes", -1)
jax.config.update("jax_persistent_cache_min_compile_time_secs", 0.0)

N_DEV = 4
S = 4096
D = 1024
H = 8
DH = 128
BLK = 512
EPS = 1e-5
SCALE = 0.08838834764831843


def _ln_mod_matmul3(x2, scale_v, shift_v, Wa, Wb, Wc):

    def body(x_ref, sc_ref, sh_ref, wa_ref, wb_ref, wc_ref, a_ref, b_ref, c_ref):
        xb = x_ref[...]
        m = jnp.mean(xb, axis=1, keepdims=True)
        xc = xb - m
        var = jnp.mean(xc * xc, axis=1, keepdims=True)
        xn = xc * lax.rsqrt(var + EPS)
        xm = (xn * (1.0 + sc_ref[...]) + sh_ref[...]).astype(jnp.bfloat16)
        a_ref[...] = (
            jnp.dot(xm, wa_ref[...], preferred_element_type=jnp.float32) * SCALE
        ).astype(jnp.bfloat16)
        b_ref[...] = jnp.dot(
            xm, wb_ref[...], preferred_element_type=jnp.float32
        ).astype(jnp.bfloat16)
        c_ref[...] = jnp.dot(
            xm, wc_ref[...], preferred_element_type=jnp.float32
        ).astype(jnp.bfloat16)

    vec_spec = pl.BlockSpec((1, D), lambda i: (0, 0))
    w_spec = pl.BlockSpec((D, D), lambda i: (0, 0))
    seq_spec = pl.BlockSpec((BLK, D), lambda i: (i, 0))
    out = jax.ShapeDtypeStruct((S, D), jnp.bfloat16)
    return pl.pallas_call(
        body,
        grid=(S // BLK,),
        in_specs=[seq_spec, vec_spec, vec_spec, w_spec, w_spec, w_spec],
        out_specs=(seq_spec, seq_spec, seq_spec),
        out_shape=(out, out, out),
    )(x2, scale_v, shift_v, Wa, Wb, Wc)


def _attention(Q, K, V):

    def body(q_ref, k_ref, v_ref, o_ref):
        q = q_ref[...]
        k = k_ref[...]
        s = lax.dot_general(
            q, k, (((1,), (1,)), ((), ())), preferred_element_type=jnp.float32
        )
        p = jnp.exp(s)
        l = jnp.sum(p, axis=1, keepdims=True)
        o = jnp.dot(
            p.astype(jnp.bfloat16), v_ref[...], preferred_element_type=jnp.float32
        )
        o_ref[...] = (o / l).astype(jnp.bfloat16)

    q_spec = pl.BlockSpec((BLK, DH), lambda h, qb: (qb, h))
    kv_spec = pl.BlockSpec((S, DH), lambda h, qb: (0, h))
    return pl.pallas_call(
        body,
        grid=(H, S // BLK),
        in_specs=[q_spec, kv_spec, kv_spec],
        out_specs=q_spec,
        out_shape=jax.ShapeDtypeStruct((S, H * DH), jnp.bfloat16),
    )(Q, K, V)


def _matmul(A, B):

    def body(a_ref, b_ref, o_ref):
        o_ref[...] = jnp.dot(
            a_ref[...], b_ref[...], preferred_element_type=jnp.float32
        ).astype(jnp.bfloat16)

    return pl.pallas_call(
        body,
        grid=(S // BLK,),
        in_specs=[
            pl.BlockSpec((BLK, D), lambda i: (i, 0)),
            pl.BlockSpec((D, D), lambda i: (0, 0)),
        ],
        out_specs=pl.BlockSpec((BLK, D), lambda i: (i, 0)),
        out_shape=jax.ShapeDtypeStruct((S, D), jnp.bfloat16),
    )(A, B)


def _ln_mod_ffn_partial(x2, scale_v, shift_v, W1, W2):

    def body(x_ref, sc_ref, sh_ref, w1_ref, w2_ref, o_ref):
        xb = x_ref[...]
        m = jnp.mean(xb, axis=1, keepdims=True)
        xc = xb - m
        var = jnp.mean(xc * xc, axis=1, keepdims=True)
        xn = xc * lax.rsqrt(var + EPS)
        xm = (xn * (1.0 + sc_ref[...]) + sh_ref[...]).astype(jnp.bfloat16)
        h = jnp.dot(xm, w1_ref[...], preferred_element_type=jnp.float32)
        h = (h * jax.nn.sigmoid(h)).astype(jnp.bfloat16)
        o_ref[...] = jnp.dot(
            h, w2_ref[...], preferred_element_type=jnp.float32
        ).astype(jnp.bfloat16)

    vec_spec = pl.BlockSpec((1, D), lambda i: (0, 0))
    w_spec = pl.BlockSpec((D, D), lambda i: (0, 0))
    seq_spec = pl.BlockSpec((BLK, D), lambda i: (i, 0))
    return pl.pallas_call(
        body,
        grid=(S // BLK,),
        in_specs=[seq_spec, vec_spec, vec_spec, w_spec, w_spec],
        out_specs=seq_spec,
        out_shape=jax.ShapeDtypeStruct((S, D), jnp.bfloat16),
    )(x2, scale_v, shift_v, W1, W2)


def _fused_attn_ar(Q, K, V, Wo):

    CH = S // (2 * N_DEV)

    def attn_rows(q_ref, k_ref, v_ref, wo_ref, row0, abuf):

        def head_body(h, carry):
            hc = pl.ds(h * DH, DH)
            q = q_ref[pl.ds(row0, CH), hc]
            k = k_ref[:, hc]
            s = lax.dot_general(
                q, k, (((1,), (1,)), ((), ())),
                preferred_element_type=jnp.float32,
            )
            p = jnp.exp(s)
            l = jnp.sum(p, axis=1, keepdims=True)
            o = jnp.dot(
                p.astype(jnp.bfloat16), v_ref[:, hc],
                preferred_element_type=jnp.float32,
            )
            abuf[:, hc] = (o / l).astype(jnp.bfloat16)
            return carry

        lax.fori_loop(0, H, head_body, 0)
        return jnp.dot(
            abuf[...], wo_ref[...], preferred_element_type=jnp.float32
        ).astype(jnp.bfloat16)

    def body(q_ref, k_ref, v_ref, wo_ref, o_ref,
             abR, abL, pbR, pbL, rbR, rbL, ssR, rsR, ssL, rsL):
        my = lax.axis_index("i")
        left = lax.rem(my + N_DEV - 1, N_DEV)
        right = lax.rem(my + 1, N_DEV)

        barrier = pltpu.get_barrier_semaphore()
        for nbr in (left, right):
            pl.semaphore_signal(
                barrier, inc=1, device_id=(nbr,),
                device_id_type=pl.DeviceIdType.MESH,
            )
        pl.semaphore_wait(barrier, 2)

        def cR(k):
            return lax.rem(my + N_DEV - k, N_DEV)

        def cL(k):
            return lax.rem(my + k, N_DEV)

        def mk_pair(srcR, dstR, srcL, dstL, step):
            rdR = pltpu.make_async_remote_copy(
                src_ref=srcR, dst_ref=dstR,
                send_sem=ssR.at[step], recv_sem=rsR.at[step],
                device_id=(right,), device_id_type=pl.DeviceIdType.MESH,
            )
            rdL = pltpu.make_async_remote_copy(
                src_ref=srcL, dst_ref=dstL,
                send_sem=ssL.at[step], recv_sem=rsL.at[step],
                device_id=(left,), device_id_type=pl.DeviceIdType.MESH,
            )
            rdR.start()
            rdL.start()
            return rdR, rdL

        descs = []
        pbR[0, :, :] = attn_rows(q_ref, k_ref, v_ref, wo_ref, cR(0) * CH, abR)
        pbL[0, :, :] = attn_rows(
            q_ref, k_ref, v_ref, wo_ref, S // 2 + cL(0) * CH, abL
        )
        descs.append(mk_pair(pbR.at[0], rbR.at[0], pbL.at[0], rbL.at[0], 0))

        for k in (1, 2, 3):
            pbR[1, :, :] = attn_rows(q_ref, k_ref, v_ref, wo_ref, cR(k) * CH, abR)
            pbL[1, :, :] = attn_rows(
                q_ref, k_ref, v_ref, wo_ref, S // 2 + cL(k) * CH, abL
            )
            rdR, rdL = descs[k - 1]
            rdR.wait_recv()
            rdL.wait_recv()
            if k < 3:
                rbR[k - 1, :, :] = rbR[k - 1, :, :] + pbR[1, :, :]
                rbL[k - 1, :, :] = rbL[k - 1, :, :] + pbL[1, :, :]
                descs.append(
                    mk_pair(rbR.at[k - 1], rbR.at[k], rbL.at[k - 1], rbL.at[k], k)
                )

        oR = cR(3)
        oL = cL(3)
        o_ref[pl.ds(oR * CH, CH), :] = rbR[2, :, :] + pbR[1, :, :]
        o_ref[pl.ds(S // 2 + oL * CH, CH), :] = rbL[2, :, :] + pbL[1, :, :]

        for rdR, rdL in descs:
            rdR.wait_send()
            rdL.wait_send()

        ag_descs = []
        for s_ag in range(3):
            gR = lax.rem(my + 1 + N_DEV - s_ag, N_DEV)
            gL = lax.rem(my + N_DEV - 1 + s_ag, N_DEV)
            rdR, rdL = mk_pair(
                o_ref.at[pl.ds(gR * CH, CH), :],
                o_ref.at[pl.ds(gR * CH, CH), :],
                o_ref.at[pl.ds(S // 2 + gL * CH, CH), :],
                o_ref.at[pl.ds(S // 2 + gL * CH, CH), :],
                3 + s_ag,
            )
            rdR.wait_recv()
            rdL.wait_recv()
            ag_descs.append((rdR, rdL))
        for rdR, rdL in ag_descs:
            rdR.wait_send()
            rdL.wait_send()

    return pl.pallas_call(
        body,
        in_specs=[pl.BlockSpec(memory_space=pltpu.VMEM)] * 4,
        out_specs=pl.BlockSpec(memory_space=pltpu.VMEM),
        out_shape=jax.ShapeDtypeStruct((S, D), jnp.bfloat16),
        scratch_shapes=[
            pltpu.VMEM((CH, D), jnp.bfloat16),
            pltpu.VMEM((CH, D), jnp.bfloat16),
            pltpu.VMEM((2, CH, D), jnp.bfloat16),
            pltpu.VMEM((2, CH, D), jnp.bfloat16),
            pltpu.VMEM((3, CH, D), jnp.bfloat16),
            pltpu.VMEM((3, CH, D), jnp.bfloat16),
            pltpu.SemaphoreType.DMA((6,)),
            pltpu.SemaphoreType.DMA((6,)),
            pltpu.SemaphoreType.DMA((6,)),
            pltpu.SemaphoreType.DMA((6,)),
        ],
        compiler_params=pltpu.CompilerParams(
            collective_id=0,
            vmem_limit_bytes=63 * 1024 * 1024,
        ),
    )(Q, K, V, Wo)


def _gated_residual(resid, gate, ar):

    def body(r_ref, g_ref, a_ref, o_ref):
        o_ref[...] = r_ref[...] + g_ref[...] * a_ref[...].astype(jnp.float32)

    return pl.pallas_call(
        body,
        grid=(S // BLK,),
        in_specs=[
            pl.BlockSpec((BLK, D), lambda i: (i, 0)),
            pl.BlockSpec((1, D), lambda i: (0, 0)),
            pl.BlockSpec((BLK, D), lambda i: (i, 0)),
        ],
        out_specs=pl.BlockSpec((BLK, D), lambda i: (i, 0)),
        out_shape=jax.ShapeDtypeStruct((S, D), jnp.float32),
    )(resid, gate, ar)


def _fused_ffn_ar(x1, scale_v, shift_v, gate, W1, W2):

    CH = S // (2 * N_DEV)

    def body(x_ref, sc_ref, sh_ref, g_ref, w1_ref, w2_ref, o_ref,
             pbR, pbL, rbR, rbL, agR, agL, ssR, rsR, ssL, rsL):
        my = lax.axis_index("i")
        left = lax.rem(my + N_DEV - 1, N_DEV)
        right = lax.rem(my + 1, N_DEV)

        barrier = pltpu.get_barrier_semaphore()
        for nbr in (left, right):
            pl.semaphore_signal(
                barrier, inc=1, device_id=(nbr,),
                device_id_type=pl.DeviceIdType.MESH,
            )
        pl.semaphore_wait(barrier, 2)

        def cR(k):
            return lax.rem(my + N_DEV - k, N_DEV)

        def cL(k):
            return lax.rem(my + k, N_DEV)

        def mk_pair(srcR, dstR, srcL, dstL, step):
            rdR = pltpu.make_async_remote_copy(
                src_ref=srcR, dst_ref=dstR,
                send_sem=ssR.at[step], recv_sem=rsR.at[step],
                device_id=(right,), device_id_type=pl.DeviceIdType.MESH,
            )
            rdL = pltpu.make_async_remote_copy(
                src_ref=srcL, dst_ref=dstL,
                send_sem=ssL.at[step], recv_sem=rsL.at[step],
                device_id=(left,), device_id_type=pl.DeviceIdType.MESH,
            )
            rdR.start()
            rdL.start()
            return rdR, rdL

        def ffn_rows(row0):
            xb = x_ref[pl.ds(row0, CH), :]
            m = jnp.mean(xb, axis=1, keepdims=True)
            xc = xb - m
            var = jnp.mean(xc * xc, axis=1, keepdims=True)
            xn = xc * lax.rsqrt(var + EPS)
            xm = (xn * (1.0 + sc_ref[...]) + sh_ref[...]).astype(jnp.bfloat16)
            h = jnp.dot(xm, w1_ref[...], preferred_element_type=jnp.float32)
            h = (h * jax.nn.sigmoid(h)).astype(jnp.bfloat16)
            return jnp.dot(
                h, w2_ref[...], preferred_element_type=jnp.float32
            ).astype(jnp.bfloat16)

        descs = []
        pbR[0, :, :] = ffn_rows(cR(0) * CH)
        pbL[0, :, :] = ffn_rows(S // 2 + cL(0) * CH)
        descs.append(mk_pair(pbR.at[0], rbR.at[0], pbL.at[0], rbL.at[0], 0))

        for k in (1, 2, 3):
            pbR[1, :, :] = ffn_rows(cR(k) * CH)
            pbL[1, :, :] = ffn_rows(S // 2 + cL(k) * CH)
            rdR, rdL = descs[k - 1]
            rdR.wait_recv()
            rdL.wait_recv()
            if k < 3:
                rbR[k - 1, :, :] = rbR[k - 1, :, :] + pbR[1, :, :]
                rbL[k - 1, :, :] = rbL[k - 1, :, :] + pbL[1, :, :]
                descs.append(
                    mk_pair(rbR.at[k - 1], rbR.at[k], rbL.at[k - 1], rbL.at[k], k)
                )

        oR = cR(3)
        oL = cL(3)
        agR[pl.ds(oR, 1), :, :] = (rbR[2, :, :] + pbR[1, :, :])[None]
        agL[pl.ds(oL, 1), :, :] = (rbL[2, :, :] + pbL[1, :, :])[None]

        for rdR, rdL in descs:
            rdR.wait_send()
            rdL.wait_send()

        ag_descs = []
        for s_ag in range(3):
            gR = lax.rem(my + 1 + N_DEV - s_ag, N_DEV)
            gL = lax.rem(my + N_DEV - 1 + s_ag, N_DEV)
            rdR, rdL = mk_pair(
                agR.at[pl.ds(gR, 1)], agR.at[pl.ds(gR, 1)],
                agL.at[pl.ds(gL, 1)], agL.at[pl.ds(gL, 1)],
                3 + s_ag,
            )
            rdR.wait_recv()
            rdL.wait_recv()
            ag_descs.append((rdR, rdL))
        for rdR, rdL in ag_descs:
            rdR.wait_send()
            rdL.wait_send()

        for c in range(N_DEV):
            rowsR = pl.ds(c * CH, CH)
            rowsL = pl.ds(S // 2 + c * CH, CH)
            o_ref[rowsR, :] = (
                x_ref[rowsR, :] + g_ref[...] * agR[c, :, :].astype(jnp.float32)
            )
            o_ref[rowsL, :] = (
                x_ref[rowsL, :] + g_ref[...] * agL[c, :, :].astype(jnp.float32)
            )

    return pl.pallas_call(
        body,
        in_specs=[pl.BlockSpec(memory_space=pltpu.VMEM)] * 6,
        out_specs=pl.BlockSpec(memory_space=pltpu.VMEM),
        out_shape=jax.ShapeDtypeStruct((S, D), jnp.float32),
        scratch_shapes=[
            pltpu.VMEM((2, CH, D), jnp.bfloat16),
            pltpu.VMEM((2, CH, D), jnp.bfloat16),
            pltpu.VMEM((3, CH, D), jnp.bfloat16),
            pltpu.VMEM((3, CH, D), jnp.bfloat16),
            pltpu.VMEM((N_DEV, CH, D), jnp.bfloat16),
            pltpu.VMEM((N_DEV, CH, D), jnp.bfloat16),
            pltpu.SemaphoreType.DMA((6,)),
            pltpu.SemaphoreType.DMA((6,)),
            pltpu.SemaphoreType.DMA((6,)),
            pltpu.SemaphoreType.DMA((6,)),
        ],
        compiler_params=pltpu.CompilerParams(
            collective_id=1,
            vmem_limit_bytes=63 * 1024 * 1024,
        ),
    )(x1, scale_v, shift_v, gate, W1, W2)


def _allreduce_residual(partial, resid, gate, collective_id):

    C = S // N_DEV
    HD = D // 2

    def body(p_ref, r_ref, g_ref, o_ref, rbR, rbL, agR, agL, ssR, rsR, ssL, rsL):
        my = lax.axis_index("i")
        left = lax.rem(my + N_DEV - 1, N_DEV)
        right = lax.rem(my + 1, N_DEV)

        barrier = pltpu.get_barrier_semaphore()
        for nbr in (left, right):
            pl.semaphore_signal(
                barrier, inc=1, device_id=(nbr,),
                device_id_type=pl.DeviceIdType.MESH,
            )
        pl.semaphore_wait(barrier, 2)

        def cR(k):
            return lax.rem(my + N_DEV - k, N_DEV)

        def cL(k):
            return lax.rem(my + k, N_DEV)

        def send_pair(srcR, dstR, srcL, dstL, step):
            rdR = pltpu.make_async_remote_copy(
                src_ref=srcR, dst_ref=dstR,
                send_sem=ssR.at[step], recv_sem=rsR.at[step],
                device_id=(right,), device_id_type=pl.DeviceIdType.MESH,
            )
            rdL = pltpu.make_async_remote_copy(
                src_ref=srcL, dst_ref=dstL,
                send_sem=ssL.at[step], recv_sem=rsL.at[step],
                device_id=(left,), device_id_type=pl.DeviceIdType.MESH,
            )
            rdR.start()
            rdL.start()
            rdR.wait()
            rdL.wait()

        send_pair(
            p_ref.at[pl.ds(cR(0) * C, C), pl.ds(0, HD)], rbR.at[0],
            p_ref.at[pl.ds(cL(0) * C, C), pl.ds(HD, HD)], rbL.at[0],
            0,
        )
        for step in (1, 2):
            rbR[step - 1, :, :] = (
                rbR[step - 1, :, :] + p_ref[pl.ds(cR(step) * C, C), pl.ds(0, HD)]
            )
            rbL[step - 1, :, :] = (
                rbL[step - 1, :, :] + p_ref[pl.ds(cL(step) * C, C), pl.ds(HD, HD)]
            )
            send_pair(rbR.at[step - 1], rbR.at[step],
                      rbL.at[step - 1], rbL.at[step], step)

        oR = cR(3)
        oL = cL(3)
        agR[pl.ds(oR, 1), :, :] = (
            rbR[2, :, :] + p_ref[pl.ds(oR * C, C), pl.ds(0, HD)]
        )[None]
        agL[pl.ds(oL, 1), :, :] = (
            rbL[2, :, :] + p_ref[pl.ds(oL * C, C), pl.ds(HD, HD)]
        )[None]

        for s_ag in range(3):
            gR = lax.rem(my + 1 + N_DEV - s_ag, N_DEV)
            gL = lax.rem(my + N_DEV - 1 + s_ag, N_DEV)
            send_pair(
                agR.at[pl.ds(gR, 1)], agR.at[pl.ds(gR, 1)],
                agL.at[pl.ds(gL, 1)], agL.at[pl.ds(gL, 1)],
                3 + s_ag,
            )

        for c in range(N_DEV):
            rows = pl.ds(c * C, C)
            o_ref[rows, pl.ds(0, HD)] = (
                r_ref[rows, pl.ds(0, HD)]
                + g_ref[:, pl.ds(0, HD)] * agR[c, :, :].astype(jnp.float32)
            )
            o_ref[rows, pl.ds(HD, HD)] = (
                r_ref[rows, pl.ds(HD, HD)]
                + g_ref[:, pl.ds(HD, HD)] * agL[c, :, :].astype(jnp.float32)
            )

    return pl.pallas_call(
        body,
        in_specs=[
            pl.BlockSpec(memory_space=pltpu.VMEM),
            pl.BlockSpec(memory_space=pltpu.VMEM),
            pl.BlockSpec(memory_space=pltpu.VMEM),
        ],
        out_specs=pl.BlockSpec(memory_space=pltpu.VMEM),
        out_shape=jax.ShapeDtypeStruct((S, D), jnp.float32),
        scratch_shapes=[
            pltpu.VMEM((3, C, HD), jnp.bfloat16),
            pltpu.VMEM((3, C, HD), jnp.bfloat16),
            pltpu.VMEM((N_DEV, C, HD), jnp.bfloat16),
            pltpu.VMEM((N_DEV, C, HD), jnp.bfloat16),
            pltpu.SemaphoreType.DMA((6,)),
            pltpu.SemaphoreType.DMA((6,)),
            pltpu.SemaphoreType.DMA((6,)),
            pltpu.SemaphoreType.DMA((6,)),
        ],
        compiler_params=pltpu.CompilerParams(
            collective_id=collective_id,
            vmem_limit_bytes=100 * 1024 * 1024,
        ),
    )(partial, resid, gate)


def kernel(x, Wq, Wk, Wv, Wo, t_emb, W_mod, W_ff1, W_ff2):
    x2 = x.reshape(S, D)

    mod = t_emb @ W_mod
    sa, sha, ga, sm, shm, gm = jnp.split(mod, 6, axis=-1)

    bf16 = jnp.bfloat16
    Wq, Wk, Wv, Wo = Wq.astype(bf16), Wk.astype(bf16), Wv.astype(bf16), Wo.astype(bf16)
    W_ff1, W_ff2 = W_ff1.astype(bf16), W_ff2.astype(bf16)

    Q, K, V = _ln_mod_matmul3(x2, sa, sha, Wq, Wk, Wv)

    ar1 = _fused_attn_ar(Q, K, V, Wo)
    x1 = _gated_residual(x2, ga, ar1)

    out = _fused_ffn_ar(x1, sm, shm, gm, W_ff1, W_ff2)

    return out.reshape(1, S, D)
